# baseline (speedup 1.0000x reference)
import functools

import jax
import jax.numpy as jnp
from jax import lax
from jax.experimental import pallas as pl
from jax.experimental.pallas import tpu as pltpu
from jax.experimental.pallas import tpu_sc as plsc

_D = 128
_N = 200
_L = 16
_TILES = 16
_LAST_OFF = 184
_NEG_BASE = 24
_LN2 = 0.6931471805599453


def _body(table, src, ctx, neg, out,
          idx, rows, part, shared, allbuf, res, sem, sem2):
    c = lax.axis_index("c")
    t = lax.axis_index("s")

    @pl.when(c == 0)
    def _core0():
        off = jnp.minimum(t * _L, _LAST_OFF)
        cp_c = pltpu.async_copy(ctx.at[pl.ds(off, _L)], idx.at[pl.ds(0, _L)], sem)
        cp_s = pltpu.async_copy(src, idx.at[pl.ds(_L, 1)], sem)
        cp_n = pltpu.async_copy(neg.at[pl.ds(off, _L)], idx.at[pl.ds(2 * _L, _L)], sem)
        cp_c.wait()
        cp_s.wait()
        cp_n.wait()
        g1 = pltpu.async_copy(table.at[idx.at[pl.ds(0, _L + 1)]],
                              rows.at[pl.ds(0, _L + 1)], sem)
        g2 = pltpu.async_copy(table.at[idx.at[pl.ds(2 * _L, _L)]],
                              rows.at[pl.ds(_NEG_BASE, _L)], sem2)
        g1.wait()

        s_chunks = [rows[_L, pl.ds(_L * k, _L)] for k in range(_D // _L)]
        lane = lax.iota(jnp.int32, _L)
        gpos = off + lane
        valid = (gpos >= t * _L) & (gpos < _N)

        zero = jnp.zeros((_L,), jnp.float32)

        csum = [zero] * (_D // _L)
        for j in range(_L):
            mj = jnp.where(off + j >= t * _L, 1.0, 0.0)
            for k in range(_D // _L):
                csum[k] = csum[k] + rows[j, pl.ds(_L * k, _L)] * mj
        pos_vec = csum[0] * s_chunks[0]
        for k in range(1, _D // _L):
            pos_vec = pos_vec + csum[k] * s_chunks[k]

        g2.wait()
        d = zero
        for j in range(_L):
            acc = rows[_NEG_BASE + j, pl.ds(0, _L)] * s_chunks[0]
            for k in range(1, _D // _L):
                acc = acc + rows[_NEG_BASE + j, pl.ds(_L * k, _L)] * s_chunks[k]
            d = jnp.where(lane == j, jnp.sum(acc), d)
        sig = 1.0 / (1.0 + jnp.exp(d))
        neg_vec = jnp.where(valid, sig, zero)

        part[pl.ds(0, _L)] = pos_vec
        part[pl.ds(_L, _L)] = neg_vec
        pltpu.sync_copy(part, shared.at[t])
        plsc.subcore_barrier()

        @pl.when(t == 0)
        def _finish():
            pltpu.sync_copy(shared, allbuf)
            pos = jnp.zeros((_L,), jnp.float32)
            negs = jnp.zeros((_L,), jnp.float32)
            for tt in range(_TILES):
                pos = pos + allbuf[tt, pl.ds(0, _L)]
                negs = negs + allbuf[tt, pl.ds(_L, _L)]
            s_tot = jnp.broadcast_to(jnp.sum(pos), (_L,))
            n_tot = jnp.broadcast_to(jnp.sum(negs), (_L,))

            y = 1.0 / (1.0 + jnp.exp(-s_tot))
            y = jnp.minimum(jnp.maximum(y, 1e-7), 1.0 - 1e-7)
            bits = plsc.bitcast(y, jnp.int32)
            x = (bits.astype(jnp.float32) * 1.1920929e-7 - 126.94269504) * _LN2
            for _ in range(3):
                x = x + y * jnp.exp(-x) - 1.0
            n_clip = jnp.minimum(jnp.maximum(n_tot, 1e-7), 1.0 - 1e-7)
            res[...] = -x - n_clip
            pltpu.sync_copy(res.at[pl.ds(0, 1)], out)


@functools.partial(jax.jit, static_argnames=())
def _sc_loss(embedding, src, ctx, neg):
    mesh = plsc.VectorSubcoreMesh(
        core_axis_name="c", subcore_axis_name="s", num_cores=1)
    f = pl.kernel(
        _body,
        out_type=jax.ShapeDtypeStruct((1,), jnp.float32),
        mesh=mesh,
        compiler_params=pltpu.CompilerParams(needs_layout_passes=False),
        scratch_types=[
            pltpu.VMEM((3 * _L,), jnp.int32),
            pltpu.VMEM((_NEG_BASE + _L, _D), jnp.float32),
            pltpu.VMEM((2 * _L,), jnp.float32),
            pltpu.VMEM_SHARED((_TILES, 2 * _L), jnp.float32),
            pltpu.VMEM((_TILES, 2 * _L), jnp.float32),
            pltpu.VMEM((_L,), jnp.float32),
            pltpu.SemaphoreType.DMA,
            pltpu.SemaphoreType.DMA,
        ],
    )
    return f(embedding, src, ctx, neg)


def kernel(embedding, source_node, context_nodes, neg_samples):
    src = source_node.astype(jnp.int32)
    ctx = context_nodes.astype(jnp.int32)
    neg = neg_samples.astype(jnp.int32)
    out = _sc_loss(embedding, src, ctx, neg)
    return jnp.reshape(out, ())


# speedup vs baseline: 1.0133x; 1.0133x over previous
import jax
import jax.numpy as jnp
from jax import lax
from jax.experimental import pallas as pl
from jax.experimental.pallas import tpu as pltpu
from jax.experimental.pallas import tpu_sc as plsc

_D = 128
_N = 200
_L = 16
_TILES = 16
_LAST_OFF = 184
_NEG_BASE = 24
_LN2 = 0.6931471805599453


def _body(table, src, ctx, neg, out,
          idx, rows, part, shared, allbuf, res, sem, sem2):
    t = lax.axis_index("s")

    off = jnp.minimum(t * _L, _LAST_OFF)
    cp_c = pltpu.async_copy(ctx.at[pl.ds(off, _L)], idx.at[pl.ds(0, _L)], sem)
    cp_s = pltpu.async_copy(src, idx.at[pl.ds(_L, 1)], sem)
    cp_n = pltpu.async_copy(neg.at[pl.ds(off, _L)], idx.at[pl.ds(2 * _L, _L)], sem)
    cp_c.wait()
    cp_s.wait()
    cp_n.wait()
    g1 = pltpu.async_copy(table.at[idx.at[pl.ds(0, _L + 1)]],
                          rows.at[pl.ds(0, _L + 1)], sem)
    g2 = pltpu.async_copy(table.at[idx.at[pl.ds(2 * _L, _L)]],
                          rows.at[pl.ds(_NEG_BASE, _L)], sem2)
    g1.wait()

    s_chunks = [rows[_L, pl.ds(_L * k, _L)] for k in range(_D // _L)]
    lane = lax.iota(jnp.int32, _L)
    gpos = off + lane
    valid = (gpos >= t * _L) & (gpos < _N)

    def dots(base):
        d = jnp.zeros((_L,), jnp.float32)
        for j in range(_L):
            acc = rows[base + j, pl.ds(0, _L)] * s_chunks[0]
            for k in range(1, _D // _L):
                acc = acc + rows[base + j, pl.ds(_L * k, _L)] * s_chunks[k]
            d = jnp.where(lane == j, jnp.sum(acc), d)
        return d

    zero = jnp.zeros((_L,), jnp.float32)
    pos_vec = jnp.where(valid, dots(0), zero)
    g2.wait()
    sig = 1.0 / (1.0 + jnp.exp(dots(_NEG_BASE)))
    neg_vec = jnp.where(valid, sig, zero)

    part[pl.ds(0, _L)] = pos_vec
    part[pl.ds(_L, _L)] = neg_vec
    pltpu.sync_copy(part, shared.at[t])
    plsc.subcore_barrier()

    @pl.when(t == 0)
    def _finish():
        pltpu.sync_copy(shared, allbuf)
        pos = jnp.zeros((_L,), jnp.float32)
        negs = jnp.zeros((_L,), jnp.float32)
        for tt in range(_TILES):
            pos = pos + allbuf[tt, pl.ds(0, _L)]
            negs = negs + allbuf[tt, pl.ds(_L, _L)]
        s_tot = jnp.broadcast_to(jnp.sum(pos), (_L,))
        n_tot = jnp.broadcast_to(jnp.sum(negs), (_L,))

        y = 1.0 / (1.0 + jnp.exp(-s_tot))
        y = jnp.minimum(jnp.maximum(y, 1e-7), 1.0 - 1e-7)
        bits = plsc.bitcast(y, jnp.int32)
        x = (bits.astype(jnp.float32) * 1.1920929e-7 - 126.94269504) * _LN2
        for _ in range(3):
            x = x + y * jnp.exp(-x) - 1.0
        n_clip = jnp.minimum(jnp.maximum(n_tot, 1e-7), 1.0 - 1e-7)
        res[...] = -x - n_clip
        pltpu.sync_copy(res.at[pl.ds(0, 1)], out)


@jax.jit
def _sc_loss(embedding, src, ctx, neg):
    mesh = plsc.VectorSubcoreMesh(
        core_axis_name="c", subcore_axis_name="s", num_cores=1)
    f = pl.kernel(
        _body,
        out_type=jax.ShapeDtypeStruct((1,), jnp.float32),
        mesh=mesh,
        compiler_params=pltpu.CompilerParams(needs_layout_passes=False),
        scratch_types=[
            pltpu.VMEM((3 * _L,), jnp.int32),
            pltpu.VMEM((_NEG_BASE + _L, _D), jnp.float32),
            pltpu.VMEM((2 * _L,), jnp.float32),
            pltpu.VMEM_SHARED((_TILES, 2 * _L), jnp.float32),
            pltpu.VMEM((_TILES, 2 * _L), jnp.float32),
            pltpu.VMEM((_L,), jnp.float32),
            pltpu.SemaphoreType.DMA,
            pltpu.SemaphoreType.DMA,
        ],
    )
    return f(embedding, src, ctx, neg)


def kernel(embedding, source_node, context_nodes, neg_samples):
    src = source_node.astype(jnp.int32)
    ctx = context_nodes.astype(jnp.int32)
    neg = neg_samples.astype(jnp.int32)
    out = _sc_loss(embedding, src, ctx, neg)
    return jnp.reshape(out, ())


# speedup vs baseline: 1.0134x; 1.0001x over previous
import jax
import jax.numpy as jnp
from jax import lax
from jax.experimental import pallas as pl
from jax.experimental.pallas import tpu as pltpu
from jax.experimental.pallas import tpu_sc as plsc

_D = 128
_N = 200
_L = 16
_TILES = 16
_LAST_OFF = 184
_NEG_BASE = 24
_LN2 = 0.6931471805599453


def _body(table, src, ctx, neg, out,
          idx, rows, part, shared, allbuf, res, sem, sem2):
    t = lax.axis_index("s")

    off = jnp.minimum(t * _L, _LAST_OFF)
    cp_c = pltpu.async_copy(ctx.at[pl.ds(off, _L)], idx.at[pl.ds(0, _L)], sem)
    cp_s = pltpu.async_copy(src, idx.at[pl.ds(_L, 1)], sem)
    cp_n = pltpu.async_copy(neg.at[pl.ds(off, _L)], idx.at[pl.ds(2 * _L, _L)], sem)
    cp_c.wait()
    cp_s.wait()
    cp_n.wait()
    g1 = pltpu.async_copy(table.at[idx.at[pl.ds(0, _L + 1)]],
                          rows.at[pl.ds(0, _L + 1)], sem)
    g2 = pltpu.async_copy(table.at[idx.at[pl.ds(2 * _L, _L)]],
                          rows.at[pl.ds(_NEG_BASE, _L)], sem2)
    g1.wait()

    s_chunks = [rows[_L, pl.ds(_L * k, _L)] for k in range(_D // _L)]
    lane = lax.iota(jnp.int32, _L)
    valid = off + lane >= t * _L

    def dots(base):
        d = jnp.zeros((_L,), jnp.float32)
        for j in range(_L):
            acc = rows[base + j, pl.ds(0, _L)] * s_chunks[0]
            for k in range(1, _D // _L):
                acc = acc + rows[base + j, pl.ds(_L * k, _L)] * s_chunks[k]
            d = jnp.where(lane == j, jnp.sum(acc), d)
        return d

    zero = jnp.zeros((_L,), jnp.float32)
    pos_vec = jnp.where(valid, dots(0), zero)
    g2.wait()
    sig = 1.0 / (1.0 + jnp.exp(dots(_NEG_BASE)))
    neg_vec = jnp.where(valid, sig, zero)

    part[pl.ds(0, _L)] = pos_vec
    part[pl.ds(_L, _L)] = neg_vec
    pltpu.sync_copy(part, shared.at[t])
    plsc.subcore_barrier()

    @pl.when(t == 0)
    def _finish():
        pltpu.sync_copy(shared, allbuf)
        pos = jnp.zeros((_L,), jnp.float32)
        negs = jnp.zeros((_L,), jnp.float32)
        for tt in range(_TILES):
            pos = pos + allbuf[tt, pl.ds(0, _L)]
            negs = negs + allbuf[tt, pl.ds(_L, _L)]
        s_tot = jnp.broadcast_to(jnp.sum(pos), (_L,))
        n_tot = jnp.broadcast_to(jnp.sum(negs), (_L,))

        y = 1.0 / (1.0 + jnp.exp(-s_tot))
        y = jnp.minimum(jnp.maximum(y, 1e-7), 1.0 - 1e-7)
        bits = plsc.bitcast(y, jnp.int32)
        x = (bits.astype(jnp.float32) * 1.1920929e-7 - 126.94269504) * _LN2
        for _ in range(3):
            x = x + y * jnp.exp(-x) - 1.0
        n_clip = jnp.minimum(jnp.maximum(n_tot, 1e-7), 1.0 - 1e-7)
        res[...] = -x - n_clip
        pltpu.sync_copy(res.at[pl.ds(0, 1)], out)


@jax.jit
def _sc_loss(embedding, src, ctx, neg):
    mesh = plsc.VectorSubcoreMesh(
        core_axis_name="c", subcore_axis_name="s", num_cores=1)
    f = pl.kernel(
        _body,
        out_type=jax.ShapeDtypeStruct((1,), jnp.float32),
        mesh=mesh,
        compiler_params=pltpu.CompilerParams(needs_layout_passes=False),
        scratch_types=[
            pltpu.VMEM((3 * _L,), jnp.int32),
            pltpu.VMEM((_NEG_BASE + _L, _D), jnp.float32),
            pltpu.VMEM((2 * _L,), jnp.float32),
            pltpu.VMEM_SHARED((_TILES, 2 * _L), jnp.float32),
            pltpu.VMEM((_TILES, 2 * _L), jnp.float32),
            pltpu.VMEM((_L,), jnp.float32),
            pltpu.SemaphoreType.DMA,
            pltpu.SemaphoreType.DMA,
        ],
    )
    return f(embedding, src, ctx, neg)


def kernel(embedding, source_node, context_nodes, neg_samples):
    src = source_node.astype(jnp.int32)
    ctx = context_nodes.astype(jnp.int32)
    neg = neg_samples.astype(jnp.int32)
    out = _sc_loss(embedding, src, ctx, neg)
    return jnp.reshape(out, ())
